# baseline (speedup 1.0000x reference)
import dataclasses
import functools

import jax
import jax.numpy as jnp
from jax import lax
from jax.experimental import pallas as pl
from jax.experimental.pallas import tpu as pltpu
from jax.experimental.pallas import tpu_sc as plsc

_N = 10000
_E = 320000
_SEQ_DIM = 9216
_NP = 10240
_BN = 512
_GRID = _NP // _BN
_NSUB = 16
_L = 16
_STR = _NP // _NSUB
_EPW = _E // 32
_UNROLL = 25
_BANKS = 3


def _gelu(t):
    return 0.5 * t * (1.0 + lax.erf(t * 0.7071067811865476))


def _dense_body(x_ref, pause_ref, seq_ref, encw_ref, fcw_ref, fcb_ref, encb_ref,
                wl_ref, wr_ref, linlb_ref, sl_ref, srb_ref):
    enc = lax.dot_general(seq_ref[...].astype(jnp.bfloat16),
                          encw_ref[...].astype(jnp.bfloat16),
                          dimension_numbers=(((1,), (1,)), ((), ())),
                          preferred_element_type=jnp.float32)
    fcw = fcw_ref[...]
    fcb = fcb_ref[...]
    wl = wl_ref[...]
    wr = wr_ref[...]
    g = _gelu(enc + encb_ref[...][None, :])
    h = x_ref[...][:, None] * fcw[None, :] + fcb[None, :] + g
    hp = pause_ref[...][:, None] * fcw[None, :] + fcb[None, :]
    fh = _gelu(h)
    fhp = _gelu(hp)
    sl_ref[...] = jnp.sum(fhp * wl[None, :32], axis=1) + jnp.sum(fh * wl[None, 32:], axis=1)
    srb_ref[...] = (jnp.sum(fhp * wr[None, :32], axis=1)
                    + jnp.sum(fh * wr[None, 32:], axis=1) + linlb_ref[0])


def _dense(x, pause, seq, encw, fcw, fcb, encb, wl, wr, linlb):
    full = lambda shape: pl.BlockSpec(shape, lambda i: tuple(0 for _ in shape))
    return pl.pallas_call(
        _dense_body,
        grid=(_GRID,),
        in_specs=[
            pl.BlockSpec((_BN,), lambda i: (i,)),
            pl.BlockSpec((_BN,), lambda i: (i,)),
            pl.BlockSpec((_BN, _SEQ_DIM), lambda i: (i, 0)),
            full((32, _SEQ_DIM)),
            full((32,)),
            full((32,)),
            full((32,)),
            full((64,)),
            full((64,)),
            full((1,)),
        ],
        out_specs=[pl.BlockSpec((_BN,), lambda i: (i,)),
                   pl.BlockSpec((_BN,), lambda i: (i,))],
        out_shape=[jax.ShapeDtypeStruct((_NP,), jnp.float32),
                   jax.ShapeDtypeStruct((_NP,), jnp.float32)],
    )(x, pause, seq, encw, fcw, fcb, encb, wl, wr, linlb)


def _sc_compiler_params():
    cp = pltpu.CompilerParams()
    if "needs_layout_passes" in pltpu.CompilerParams.__dataclass_fields__:
        cp = dataclasses.replace(cp, needs_layout_passes=False)
    return cp


_MESH = plsc.VectorSubcoreMesh(core_axis_name="c", subcore_axis_name="s")


def _stripe_reduce(part_all, stag, stripe, sid, sem):
    base = sid * _STR
    handles = [pltpu.async_copy(part_all.at[j, pl.ds(base, _STR)], stag.at[j], sem)
               for j in range(_NSUB)]
    for h in handles:
        h.wait()

    @pl.loop(0, _STR, step=_L)
    def _(k2):
        v = stag[0, pl.ds(k2, _L)]
        for j in range(1, _NSUB):
            v = v + stag[j, pl.ds(k2, _L)]
        stripe[pl.ds(k2, _L)] = v


def _edges(sl, edge_index):
    @functools.partial(
        pl.kernel,
        mesh=_MESH,
        compiler_params=_sc_compiler_params(),
        out_type=[jax.ShapeDtypeStruct((2 * _NP,), jnp.float32),
                  jax.ShapeDtypeStruct((2 * _NP,), jnp.float32)],
        scratch_types=[
            pltpu.VMEM((_NP,), jnp.float32),
            pltpu.VMEM((_NP,), jnp.float32),
            pltpu.VMEM((_NP,), jnp.float32),
            pltpu.VMEM((_NP,), jnp.float32),
            pltpu.VMEM((_NP,), jnp.float32),
            pltpu.VMEM((_NP,), jnp.float32),
            pltpu.VMEM((_NP,), jnp.float32),
            pltpu.VMEM((_EPW,), jnp.int32),
            pltpu.VMEM((_EPW,), jnp.int32),
            pltpu.VMEM((_NSUB, _STR), jnp.float32),
            pltpu.VMEM((_STR,), jnp.float32),
            pltpu.VMEM_SHARED((_NSUB, _NP), jnp.float32),
            pltpu.VMEM_SHARED((_NSUB, _NP), jnp.float32),
            pltpu.SemaphoreType.DMA,
        ],
    )
    def k(sl_hbm, e_hbm, acc_out, cnt_out,
          table, acc0, acc1, acc2, cnt0, cnt1, cnt2,
          srcb, dstb, stag, stripe, acc_all, cnt_all, sem):
        accb = (acc0, acc1, acc2)
        cntb = (cnt0, cnt1, cnt2)
        cid = lax.axis_index("c")
        sid = lax.axis_index("s")
        zv = jnp.zeros((_L,), jnp.float32)
        ones = jnp.ones((_L,), jnp.float32)
        ebase = (cid * _NSUB + sid) * _EPW
        h_tab = pltpu.async_copy(sl_hbm, table, sem)
        h_src = pltpu.async_copy(e_hbm.at[pl.ds(ebase, _EPW)], srcb, sem)
        h_dst = pltpu.async_copy(e_hbm.at[pl.ds(_E + ebase, _EPW)], dstb, sem)

        @pl.loop(0, _NP, step=_L)
        def _(i):
            for b in range(_BANKS):
                accb[b][pl.ds(i, _L)] = zv
                cntb[b][pl.ds(i, _L)] = zv

        h_tab.wait()
        h_src.wait()
        h_dst.wait()

        @pl.loop(0, _EPW, step=_L * _UNROLL)
        def _(i):
            for u in range(_UNROLL):
                sv = srcb[pl.ds(i + u * _L, _L)]
                dv = dstb[pl.ds(i + u * _L, _L)]
                vals = plsc.load_gather(table, [sv])
                b = u % _BANKS
                plsc.addupdate_scatter(accb[b], [dv], vals)
                plsc.addupdate_scatter(cntb[b], [dv], ones)

        @pl.loop(0, _NP, step=_L)
        def _(i):
            s = pl.ds(i, _L)
            acc0[s] = acc0[s] + acc1[s] + acc2[s]
            cnt0[s] = cnt0[s] + cnt1[s] + cnt2[s]

        pltpu.sync_copy(acc0, acc_all.at[sid])
        pltpu.sync_copy(cnt0, cnt_all.at[sid])
        plsc.subcore_barrier()

        _stripe_reduce(acc_all, stag, stripe, sid, sem)
        pltpu.sync_copy(stripe, acc_out.at[pl.ds(cid * _NP + sid * _STR, _STR)])
        _stripe_reduce(cnt_all, stag, stripe, sid, sem)
        pltpu.sync_copy(stripe, cnt_out.at[pl.ds(cid * _NP + sid * _STR, _STR)])

    return k(sl, edge_index)


_FBN = 128
_FGRID = 79


def _final_body(acc_lo, acc_hi, cnt_lo, cnt_hi, srb_ref, out_ref):
    cnt = jnp.maximum(cnt_lo[...] + cnt_hi[...], 1.0)
    v = (acc_lo[...] + acc_hi[...]) / cnt + srb_ref[...]
    out_ref[...] = v.reshape(_FBN, 1)


def _finalize(acc2, cnt2, srb):
    blk = lambda off: pl.BlockSpec((_FBN,), lambda i, off=off: (i + off,))
    return pl.pallas_call(
        _final_body,
        grid=(_FGRID,),
        in_specs=[blk(0), blk(_NP // _FBN), blk(0), blk(_NP // _FBN), blk(0)],
        out_specs=pl.BlockSpec((_FBN, 1), lambda i: (i, 0)),
        out_shape=jax.ShapeDtypeStruct((_N, 1), jnp.float32),
    )(acc2, acc2, cnt2, cnt2, srb)


def kernel(x, seq, pause, edge_index, fc_W, fc_b, enc_W, enc_b, lin_l_W, lin_l_b, lin_r_W):
    ei = edge_index.astype(jnp.int32).reshape(2 * _E)
    sl, srb = _dense(x, pause, seq, enc_W, fc_W.reshape(32), fc_b, enc_b,
                     lin_l_W.reshape(64), lin_r_W.reshape(64), lin_l_b)
    acc2, cnt2 = _edges(sl, ei)
    return _finalize(acc2, cnt2, srb)


# speedup vs baseline: 1.0037x; 1.0037x over previous
import dataclasses
import functools

import jax
import jax.numpy as jnp
from jax import lax
from jax.experimental import pallas as pl
from jax.experimental.pallas import tpu as pltpu
from jax.experimental.pallas import tpu_sc as plsc

_N = 10000
_E = 320000
_SEQ_DIM = 9216
_NP = 10240
_BN = 512
_GRID = _NP // _BN
_NSUB = 16
_L = 16
_STR = _NP // _NSUB
_EPW = _E // 32
_UNROLL = 5
_BANKS = 3


def _gelu(t):
    return 0.5 * t * (1.0 + lax.erf(t * 0.7071067811865476))


def _dense_body(x_ref, pause_ref, seq_ref, encw_ref, fcw_ref, fcb_ref, encb_ref,
                wl_ref, wr_ref, linlb_ref, sl_ref, srb_ref):
    enc = lax.dot_general(seq_ref[...].astype(jnp.bfloat16),
                          encw_ref[...].astype(jnp.bfloat16),
                          dimension_numbers=(((1,), (1,)), ((), ())),
                          preferred_element_type=jnp.float32)
    fcw = fcw_ref[...]
    fcb = fcb_ref[...]
    wl = wl_ref[...]
    wr = wr_ref[...]
    g = _gelu(enc + encb_ref[...][None, :])
    h = x_ref[...][:, None] * fcw[None, :] + fcb[None, :] + g
    hp = pause_ref[...][:, None] * fcw[None, :] + fcb[None, :]
    fh = _gelu(h)
    fhp = _gelu(hp)
    sl_ref[...] = jnp.sum(fhp * wl[None, :32], axis=1) + jnp.sum(fh * wl[None, 32:], axis=1)
    srb_ref[...] = (jnp.sum(fhp * wr[None, :32], axis=1)
                    + jnp.sum(fh * wr[None, 32:], axis=1) + linlb_ref[0])


def _dense(x, pause, seq, encw, fcw, fcb, encb, wl, wr, linlb):
    full = lambda shape: pl.BlockSpec(shape, lambda i: tuple(0 for _ in shape))
    return pl.pallas_call(
        _dense_body,
        grid=(_GRID,),
        in_specs=[
            pl.BlockSpec((_BN,), lambda i: (i,)),
            pl.BlockSpec((_BN,), lambda i: (i,)),
            pl.BlockSpec((_BN, _SEQ_DIM), lambda i: (i, 0)),
            full((32, _SEQ_DIM)),
            full((32,)),
            full((32,)),
            full((32,)),
            full((64,)),
            full((64,)),
            full((1,)),
        ],
        out_specs=[pl.BlockSpec((_BN,), lambda i: (i,)),
                   pl.BlockSpec((_BN,), lambda i: (i,))],
        out_shape=[jax.ShapeDtypeStruct((_NP,), jnp.float32),
                   jax.ShapeDtypeStruct((_NP,), jnp.float32)],
    )(x, pause, seq, encw, fcw, fcb, encb, wl, wr, linlb)


def _sc_compiler_params():
    cp = pltpu.CompilerParams()
    if "needs_layout_passes" in pltpu.CompilerParams.__dataclass_fields__:
        cp = dataclasses.replace(cp, needs_layout_passes=False)
    return cp


_MESH = plsc.VectorSubcoreMesh(core_axis_name="c", subcore_axis_name="s")


def _stripe_reduce(part_all, stag, stripe, sid, sem):
    base = sid * _STR
    handles = [pltpu.async_copy(part_all.at[j, pl.ds(base, _STR)], stag.at[j], sem)
               for j in range(_NSUB)]
    for h in handles:
        h.wait()

    @pl.loop(0, _STR, step=_L)
    def _(k2):
        v = stag[0, pl.ds(k2, _L)]
        for j in range(1, _NSUB):
            v = v + stag[j, pl.ds(k2, _L)]
        stripe[pl.ds(k2, _L)] = v


def _edges(sl, edge_index):
    @functools.partial(
        pl.kernel,
        mesh=_MESH,
        compiler_params=_sc_compiler_params(),
        out_type=[jax.ShapeDtypeStruct((2 * _NP,), jnp.float32),
                  jax.ShapeDtypeStruct((2 * _NP,), jnp.float32)],
        scratch_types=[
            pltpu.VMEM((_NP,), jnp.float32),
            pltpu.VMEM((_NP,), jnp.float32),
            pltpu.VMEM((_NP,), jnp.float32),
            pltpu.VMEM((_NP,), jnp.float32),
            pltpu.VMEM((_NP,), jnp.float32),
            pltpu.VMEM((_NP,), jnp.float32),
            pltpu.VMEM((_NP,), jnp.float32),
            pltpu.VMEM((_EPW,), jnp.int32),
            pltpu.VMEM((_EPW,), jnp.int32),
            pltpu.VMEM((_NSUB, _STR), jnp.float32),
            pltpu.VMEM((_STR,), jnp.float32),
            pltpu.VMEM_SHARED((_NSUB, _NP), jnp.float32),
            pltpu.VMEM_SHARED((_NSUB, _NP), jnp.float32),
            pltpu.SemaphoreType.DMA,
        ],
    )
    def k(sl_hbm, e_hbm, acc_out, cnt_out,
          table, acc0, acc1, acc2, cnt0, cnt1, cnt2,
          srcb, dstb, stag, stripe, acc_all, cnt_all, sem):
        accb = (acc0, acc1, acc2)
        cntb = (cnt0, cnt1, cnt2)
        cid = lax.axis_index("c")
        sid = lax.axis_index("s")
        zv = jnp.zeros((_L,), jnp.float32)
        ones = jnp.ones((_L,), jnp.float32)
        ebase = (cid * _NSUB + sid) * _EPW
        h_tab = pltpu.async_copy(sl_hbm, table, sem)
        h_src = pltpu.async_copy(e_hbm.at[pl.ds(ebase, _EPW)], srcb, sem)
        h_dst = pltpu.async_copy(e_hbm.at[pl.ds(_E + ebase, _EPW)], dstb, sem)

        @pl.loop(0, _NP, step=_L)
        def _(i):
            for b in range(_BANKS):
                accb[b][pl.ds(i, _L)] = zv
                cntb[b][pl.ds(i, _L)] = zv

        h_tab.wait()
        h_src.wait()
        h_dst.wait()

        @pl.loop(0, _EPW, step=_L * _UNROLL)
        def _(i):
            for u in range(_UNROLL):
                sv = srcb[pl.ds(i + u * _L, _L)]
                dv = dstb[pl.ds(i + u * _L, _L)]
                vals = plsc.load_gather(table, [sv])
                b = u % _BANKS
                plsc.addupdate_scatter(accb[b], [dv], vals)
                plsc.addupdate_scatter(cntb[b], [dv], ones)

        @pl.loop(0, _NP, step=_L)
        def _(i):
            s = pl.ds(i, _L)
            acc0[s] = acc0[s] + acc1[s] + acc2[s]
            cnt0[s] = cnt0[s] + cnt1[s] + cnt2[s]

        pltpu.sync_copy(acc0, acc_all.at[sid])
        pltpu.sync_copy(cnt0, cnt_all.at[sid])
        plsc.subcore_barrier()

        _stripe_reduce(acc_all, stag, stripe, sid, sem)
        pltpu.sync_copy(stripe, acc_out.at[pl.ds(cid * _NP + sid * _STR, _STR)])
        _stripe_reduce(cnt_all, stag, stripe, sid, sem)
        pltpu.sync_copy(stripe, cnt_out.at[pl.ds(cid * _NP + sid * _STR, _STR)])

    return k(sl, edge_index)


_FBN = 128
_FGRID = 79


def _final_body(acc_lo, acc_hi, cnt_lo, cnt_hi, srb_ref, out_ref):
    cnt = jnp.maximum(cnt_lo[...] + cnt_hi[...], 1.0)
    v = (acc_lo[...] + acc_hi[...]) / cnt + srb_ref[...]
    out_ref[...] = v.reshape(_FBN, 1)


def _finalize(acc2, cnt2, srb):
    blk = lambda off: pl.BlockSpec((_FBN,), lambda i, off=off: (i + off,))
    return pl.pallas_call(
        _final_body,
        grid=(_FGRID,),
        in_specs=[blk(0), blk(_NP // _FBN), blk(0), blk(_NP // _FBN), blk(0)],
        out_specs=pl.BlockSpec((_FBN, 1), lambda i: (i, 0)),
        out_shape=jax.ShapeDtypeStruct((_N, 1), jnp.float32),
    )(acc2, acc2, cnt2, cnt2, srb)


def kernel(x, seq, pause, edge_index, fc_W, fc_b, enc_W, enc_b, lin_l_W, lin_l_b, lin_r_W):
    ei = edge_index.astype(jnp.int32).reshape(2 * _E)
    sl, srb = _dense(x, pause, seq, enc_W, fc_W.reshape(32), fc_b, enc_b,
                     lin_l_W.reshape(64), lin_r_W.reshape(64), lin_l_b)
    acc2, cnt2 = _edges(sl, ei)
    return _finalize(acc2, cnt2, srb)


# speedup vs baseline: 1.1740x; 1.1697x over previous
import dataclasses
import functools

import jax
import jax.numpy as jnp
from jax import lax
from jax.experimental import pallas as pl
from jax.experimental.pallas import tpu as pltpu
from jax.experimental.pallas import tpu_sc as plsc

_N = 10000
_E = 320000
_SEQ_DIM = 9216
_NP = 10240
_BN = 512
_GRID = _NP // _BN
_NSUB = 16
_L = 16
_STR = _NP // _NSUB
_EPW = _E // 32
_UNROLL = 5
_BANKS = 3


def _gelu(t):
    return 0.5 * t * (1.0 + lax.erf(t * 0.7071067811865476))


def _dense_body(x_ref, pause_ref, seq_ref, encw_ref, fcw_ref, fcb_ref, encb_ref,
                wl_ref, wr_ref, linlb_ref, sl_ref, srb_ref):
    enc = lax.dot_general(seq_ref[...].astype(jnp.bfloat16),
                          encw_ref[...].astype(jnp.bfloat16),
                          dimension_numbers=(((1,), (1,)), ((), ())),
                          preferred_element_type=jnp.float32)
    fcw = fcw_ref[...]
    fcb = fcb_ref[...]
    wl = wl_ref[...]
    wr = wr_ref[...]
    g = _gelu(enc + encb_ref[...][None, :])
    h = x_ref[...][:, None] * fcw[None, :] + fcb[None, :] + g
    hp = pause_ref[...][:, None] * fcw[None, :] + fcb[None, :]
    fh = _gelu(h)
    fhp = _gelu(hp)
    sl_ref[...] = jnp.sum(fhp * wl[None, :32], axis=1) + jnp.sum(fh * wl[None, 32:], axis=1)
    srb_ref[...] = (jnp.sum(fhp * wr[None, :32], axis=1)
                    + jnp.sum(fh * wr[None, 32:], axis=1) + linlb_ref[0])


def _dense(x, pause, seq, encw, fcw, fcb, encb, wl, wr, linlb):
    full = lambda shape: pl.BlockSpec(shape, lambda i: tuple(0 for _ in shape))
    return pl.pallas_call(
        _dense_body,
        grid=(_GRID,),
        in_specs=[
            pl.BlockSpec((_BN,), lambda i: (i,)),
            pl.BlockSpec((_BN,), lambda i: (i,)),
            pl.BlockSpec((_BN, _SEQ_DIM), lambda i: (i, 0)),
            full((32, _SEQ_DIM)),
            full((32,)),
            full((32,)),
            full((32,)),
            full((64,)),
            full((64,)),
            full((1,)),
        ],
        out_specs=[pl.BlockSpec((_BN,), lambda i: (i,)),
                   pl.BlockSpec((_BN,), lambda i: (i,))],
        out_shape=[jax.ShapeDtypeStruct((_NP,), jnp.float32),
                   jax.ShapeDtypeStruct((_NP,), jnp.float32)],
    )(x, pause, seq, encw, fcw, fcb, encb, wl, wr, linlb)


def _sc_compiler_params():
    cp = pltpu.CompilerParams()
    if "needs_layout_passes" in pltpu.CompilerParams.__dataclass_fields__:
        cp = dataclasses.replace(cp, needs_layout_passes=False)
    return cp


_MESH = plsc.VectorSubcoreMesh(core_axis_name="c", subcore_axis_name="s")


def _stripe_reduce(part_all, stag, stripe, sid, sem):
    base = sid * _STR
    handles = [pltpu.async_copy(part_all.at[j, pl.ds(base, _STR)], stag.at[j], sem)
               for j in range(_NSUB)]
    for h in handles:
        h.wait()

    @pl.loop(0, _STR, step=_L)
    def _(k2):
        v = stag[0, pl.ds(k2, _L)]
        for j in range(1, _NSUB):
            v = v + stag[j, pl.ds(k2, _L)]
        stripe[pl.ds(k2, _L)] = v


def _edges(sl, src, dst):
    @functools.partial(
        pl.kernel,
        mesh=_MESH,
        compiler_params=_sc_compiler_params(),
        out_type=[jax.ShapeDtypeStruct((2 * _NP,), jnp.float32),
                  jax.ShapeDtypeStruct((2 * _NP,), jnp.float32)],
        scratch_types=[
            pltpu.VMEM((_NP,), jnp.float32),
            pltpu.VMEM((_NP,), jnp.float32),
            pltpu.VMEM((_NP,), jnp.float32),
            pltpu.VMEM((_NP,), jnp.float32),
            pltpu.VMEM((_NP,), jnp.float32),
            pltpu.VMEM((_NP,), jnp.float32),
            pltpu.VMEM((_NP,), jnp.float32),
            pltpu.VMEM((_EPW,), jnp.int32),
            pltpu.VMEM((_EPW,), jnp.int32),
            pltpu.VMEM((_NSUB, _STR), jnp.float32),
            pltpu.VMEM((_STR,), jnp.float32),
            pltpu.VMEM_SHARED((_NSUB, _NP), jnp.float32),
            pltpu.VMEM_SHARED((_NSUB, _NP), jnp.float32),
            pltpu.VMEM_SHARED((_NP,), jnp.float32),
            pltpu.SemaphoreType.DMA,
        ],
    )
    def k(sl_hbm, src_hbm, dst_hbm, acc_out, cnt_out,
          table, acc0, acc1, acc2, cnt0, cnt1, cnt2,
          srcb, dstb, stag, stripe, acc_all, cnt_all, sl_sh, sem):
        accb = (acc0, acc1, acc2)
        cntb = (cnt0, cnt1, cnt2)
        cid = lax.axis_index("c")
        sid = lax.axis_index("s")
        zv = jnp.zeros((_L,), jnp.float32)
        ones = jnp.ones((_L,), jnp.float32)
        ebase = (cid * _NSUB + sid) * _EPW
        h_tab = pltpu.async_copy(sl_hbm.at[pl.ds(sid * _STR, _STR)],
                                 sl_sh.at[pl.ds(sid * _STR, _STR)], sem)
        h_src = pltpu.async_copy(src_hbm.at[pl.ds(ebase, _EPW)], srcb, sem)
        h_dst = pltpu.async_copy(dst_hbm.at[pl.ds(ebase, _EPW)], dstb, sem)

        @pl.loop(0, _NP, step=_L)
        def _(i):
            for b in range(_BANKS):
                accb[b][pl.ds(i, _L)] = zv
                cntb[b][pl.ds(i, _L)] = zv

        h_tab.wait()
        plsc.subcore_barrier()
        h_sh = pltpu.async_copy(sl_sh, table, sem)
        h_src.wait()
        h_dst.wait()
        h_sh.wait()

        @pl.loop(0, _EPW, step=_L * _UNROLL)
        def _(i):
            for u in range(_UNROLL):
                sv = srcb[pl.ds(i + u * _L, _L)]
                dv = dstb[pl.ds(i + u * _L, _L)]
                vals = plsc.load_gather(table, [sv])
                b = u % _BANKS
                plsc.addupdate_scatter(accb[b], [dv], vals)
                plsc.addupdate_scatter(cntb[b], [dv], ones)

        @pl.loop(0, _NP, step=_L * 4)
        def _(i):
            for u in range(4):
                s = pl.ds(i + u * _L, _L)
                acc0[s] = acc0[s] + acc1[s] + acc2[s]
                cnt0[s] = cnt0[s] + cnt1[s] + cnt2[s]

        pltpu.sync_copy(acc0, acc_all.at[sid])
        pltpu.sync_copy(cnt0, cnt_all.at[sid])
        plsc.subcore_barrier()

        _stripe_reduce(acc_all, stag, stripe, sid, sem)
        pltpu.sync_copy(stripe, acc_out.at[pl.ds(cid * _NP + sid * _STR, _STR)])
        _stripe_reduce(cnt_all, stag, stripe, sid, sem)
        pltpu.sync_copy(stripe, cnt_out.at[pl.ds(cid * _NP + sid * _STR, _STR)])

    return k(sl, src, dst)


_FSTR = 320


def _finalize(acc2, cnt2, srb):
    @functools.partial(
        pl.kernel,
        mesh=_MESH,
        compiler_params=_sc_compiler_params(),
        out_type=jax.ShapeDtypeStruct((_N,), jnp.float32),
        scratch_types=[
            pltpu.VMEM((_FSTR,), jnp.float32),
            pltpu.VMEM((_FSTR,), jnp.float32),
            pltpu.VMEM((_FSTR,), jnp.float32),
            pltpu.VMEM((_FSTR,), jnp.float32),
            pltpu.VMEM((_FSTR,), jnp.float32),
            pltpu.SemaphoreType.DMA,
        ],
    )
    def k(acc_hbm, cnt_hbm, srb_hbm, out_hbm, a, b, c, d, e, sem):
        cid = lax.axis_index("c")
        sid = lax.axis_index("s")
        wid = cid * _NSUB + sid
        base = wid * _FSTR

        def run(size):
            sl_lo = pl.ds(base, size)
            sl_hi = pl.ds(_NP + base, size)
            handles = [
                pltpu.async_copy(acc_hbm.at[sl_lo], a.at[pl.ds(0, size)], sem),
                pltpu.async_copy(acc_hbm.at[sl_hi], b.at[pl.ds(0, size)], sem),
                pltpu.async_copy(cnt_hbm.at[sl_lo], c.at[pl.ds(0, size)], sem),
                pltpu.async_copy(cnt_hbm.at[sl_hi], d.at[pl.ds(0, size)], sem),
                pltpu.async_copy(srb_hbm.at[sl_lo], e.at[pl.ds(0, size)], sem),
            ]
            for h in handles:
                h.wait()

            @pl.loop(0, size, step=_L)
            def _(i):
                s = pl.ds(i, _L)
                cnt = jnp.maximum(c[s] + d[s], 1.0)
                a[s] = (a[s] + b[s]) / cnt + e[s]

            pltpu.sync_copy(a.at[pl.ds(0, size)], out_hbm.at[pl.ds(base, size)])

        @pl.when(wid < 31)
        def _():
            run(_FSTR)

        @pl.when(wid == 31)
        def _():
            run(_N - 31 * _FSTR)

    return k(acc2, cnt2, srb)


def kernel(x, seq, pause, edge_index, fc_W, fc_b, enc_W, enc_b, lin_l_W, lin_l_b, lin_r_W):
    ei = edge_index.astype(jnp.int32)
    sl, srb = _dense(x, pause, seq, enc_W, fc_W.reshape(32), fc_b, enc_b,
                     lin_l_W.reshape(64), lin_r_W.reshape(64), lin_l_b)
    acc2, cnt2 = _edges(sl, ei[0], ei[1])
    out = _finalize(acc2, cnt2, srb)
    return out.reshape(_N, 1)


# speedup vs baseline: 1.2507x; 1.0653x over previous
import dataclasses
import functools

import jax
import jax.numpy as jnp
from jax import lax
from jax.experimental import pallas as pl
from jax.experimental.pallas import tpu as pltpu
from jax.experimental.pallas import tpu_sc as plsc

_N = 10000
_E = 320000
_SEQ_DIM = 9216
_NP = 10240
_BN = 512
_GRID = _NP // _BN
_NSUB = 16
_L = 16
_STR = _NP // _NSUB
_EPW = _E // 32
_UNROLL = 5
_BANKS = 3


def _gelu(t):
    return 0.5 * t * (1.0 + lax.erf(t * 0.7071067811865476))


def _dense_body(x_ref, pause_ref, seq_ref, encw_ref, fcw_ref, fcb_ref, encb_ref,
                wl_ref, wr_ref, linlb_ref, sl_ref, srb_ref):
    enc = lax.dot_general(seq_ref[...].astype(jnp.bfloat16),
                          encw_ref[...].astype(jnp.bfloat16),
                          dimension_numbers=(((1,), (1,)), ((), ())),
                          preferred_element_type=jnp.float32)
    fcw = fcw_ref[...]
    fcb = fcb_ref[...]
    wl = wl_ref[...]
    wr = wr_ref[...]
    g = _gelu(enc + encb_ref[...][None, :])
    h = x_ref[...][:, None] * fcw[None, :] + fcb[None, :] + g
    hp = pause_ref[...][:, None] * fcw[None, :] + fcb[None, :]
    fh = _gelu(h)
    fhp = _gelu(hp)
    sl_ref[...] = jnp.sum(fhp * wl[None, :32], axis=1) + jnp.sum(fh * wl[None, 32:], axis=1)
    srb_ref[...] = (jnp.sum(fhp * wr[None, :32], axis=1)
                    + jnp.sum(fh * wr[None, 32:], axis=1) + linlb_ref[0])


def _dense(x, pause, seq, encw, fcw, fcb, encb, wl, wr, linlb):
    full = lambda shape: pl.BlockSpec(shape, lambda i: tuple(0 for _ in shape))
    return pl.pallas_call(
        _dense_body,
        grid=(_GRID,),
        in_specs=[
            pl.BlockSpec((_BN,), lambda i: (i,)),
            pl.BlockSpec((_BN,), lambda i: (i,)),
            pl.BlockSpec((_BN, _SEQ_DIM), lambda i: (i, 0)),
            full((32, _SEQ_DIM)),
            full((32,)),
            full((32,)),
            full((32,)),
            full((64,)),
            full((64,)),
            full((1,)),
        ],
        out_specs=[pl.BlockSpec((_BN,), lambda i: (i,)),
                   pl.BlockSpec((_BN,), lambda i: (i,))],
        out_shape=[jax.ShapeDtypeStruct((_NP,), jnp.float32),
                   jax.ShapeDtypeStruct((_NP,), jnp.float32)],
    )(x, pause, seq, encw, fcw, fcb, encb, wl, wr, linlb)


def _sc_compiler_params():
    cp = pltpu.CompilerParams()
    if "needs_layout_passes" in pltpu.CompilerParams.__dataclass_fields__:
        cp = dataclasses.replace(cp, needs_layout_passes=False)
    return cp


_MESH = plsc.VectorSubcoreMesh(core_axis_name="c", subcore_axis_name="s")


def _stripe_reduce(part_all, stag, stripe, sid, sem):
    base = sid * _STR
    handles = [pltpu.async_copy(part_all.at[j, pl.ds(base, _STR)], stag.at[j], sem)
               for j in range(_NSUB)]
    for h in handles:
        h.wait()

    @pl.loop(0, _STR, step=_L)
    def _(k2):
        v = stag[0, pl.ds(k2, _L)]
        for j in range(1, _NSUB):
            v = v + stag[j, pl.ds(k2, _L)]
        stripe[pl.ds(k2, _L)] = v


def _edges(sl, eflat):
    @functools.partial(
        pl.kernel,
        mesh=_MESH,
        compiler_params=_sc_compiler_params(),
        out_type=[jax.ShapeDtypeStruct((2 * _NP,), jnp.float32),
                  jax.ShapeDtypeStruct((2 * _NP,), jnp.float32)],
        scratch_types=[
            pltpu.VMEM((_NP,), jnp.float32),
            pltpu.VMEM((_NP,), jnp.float32),
            pltpu.VMEM((_NP,), jnp.float32),
            pltpu.VMEM((_NP,), jnp.float32),
            pltpu.VMEM((_NP,), jnp.float32),
            pltpu.VMEM((_NP,), jnp.float32),
            pltpu.VMEM((_NP,), jnp.float32),
            pltpu.VMEM((_EPW,), jnp.int32),
            pltpu.VMEM((_EPW,), jnp.int32),
            pltpu.VMEM((_NSUB, _STR), jnp.float32),
            pltpu.VMEM((_STR,), jnp.float32),
            pltpu.VMEM_SHARED((_NSUB, _NP), jnp.float32),
            pltpu.VMEM_SHARED((_NSUB, _NP), jnp.float32),
            pltpu.VMEM_SHARED((_NP,), jnp.float32),
            pltpu.SemaphoreType.DMA,
        ],
    )
    def k(sl_hbm, e_hbm, acc_out, cnt_out,
          table, acc0, acc1, acc2, cnt0, cnt1, cnt2,
          srcb, dstb, stag, stripe, acc_all, cnt_all, sl_sh, sem):
        accb = (acc0, acc1, acc2)
        cntb = (cnt0, cnt1, cnt2)
        cid = lax.axis_index("c")
        sid = lax.axis_index("s")
        zv = jnp.zeros((_L,), jnp.float32)
        ones = jnp.ones((_L,), jnp.float32)
        ebase = (cid * _NSUB + sid) * _EPW
        h_tab = pltpu.async_copy(sl_hbm.at[pl.ds(sid * _STR, _STR)],
                                 sl_sh.at[pl.ds(sid * _STR, _STR)], sem)
        h_src = pltpu.async_copy(e_hbm.at[pl.ds(ebase, _EPW)], srcb, sem)
        h_dst = pltpu.async_copy(e_hbm.at[pl.ds(_E + ebase, _EPW)], dstb, sem)

        @pl.loop(0, _NP, step=_L)
        def _(i):
            for b in range(_BANKS):
                accb[b][pl.ds(i, _L)] = zv
                cntb[b][pl.ds(i, _L)] = zv

        h_tab.wait()
        plsc.subcore_barrier()
        h_sh = pltpu.async_copy(sl_sh, table, sem)
        h_src.wait()
        h_dst.wait()
        h_sh.wait()

        @pl.loop(0, _EPW, step=_L * _UNROLL)
        def _(i):
            for u in range(_UNROLL):
                sv = srcb[pl.ds(i + u * _L, _L)]
                dv = dstb[pl.ds(i + u * _L, _L)]
                vals = plsc.load_gather(table, [sv])
                b = u % _BANKS
                plsc.addupdate_scatter(accb[b], [dv], vals)
                plsc.addupdate_scatter(cntb[b], [dv], ones)

        @pl.loop(0, _NP, step=_L * 4)
        def _(i):
            for u in range(4):
                s = pl.ds(i + u * _L, _L)
                acc0[s] = acc0[s] + acc1[s] + acc2[s]
                cnt0[s] = cnt0[s] + cnt1[s] + cnt2[s]

        pltpu.sync_copy(acc0, acc_all.at[sid])
        pltpu.sync_copy(cnt0, cnt_all.at[sid])
        plsc.subcore_barrier()

        _stripe_reduce(acc_all, stag, stripe, sid, sem)
        pltpu.sync_copy(stripe, acc_out.at[pl.ds(cid * _NP + sid * _STR, _STR)])
        _stripe_reduce(cnt_all, stag, stripe, sid, sem)
        pltpu.sync_copy(stripe, cnt_out.at[pl.ds(cid * _NP + sid * _STR, _STR)])

    return k(sl, eflat)


_FSTR = 320


def _finalize(acc2, cnt2, srb):
    @functools.partial(
        pl.kernel,
        mesh=_MESH,
        compiler_params=_sc_compiler_params(),
        out_type=jax.ShapeDtypeStruct((_N,), jnp.float32),
        scratch_types=[
            pltpu.VMEM((_FSTR,), jnp.float32),
            pltpu.VMEM((_FSTR,), jnp.float32),
            pltpu.VMEM((_FSTR,), jnp.float32),
            pltpu.VMEM((_FSTR,), jnp.float32),
            pltpu.VMEM((_FSTR,), jnp.float32),
            pltpu.SemaphoreType.DMA,
        ],
    )
    def k(acc_hbm, cnt_hbm, srb_hbm, out_hbm, a, b, c, d, e, sem):
        cid = lax.axis_index("c")
        sid = lax.axis_index("s")
        wid = cid * _NSUB + sid
        base = wid * _FSTR

        def run(size):
            sl_lo = pl.ds(base, size)
            sl_hi = pl.ds(_NP + base, size)
            handles = [
                pltpu.async_copy(acc_hbm.at[sl_lo], a.at[pl.ds(0, size)], sem),
                pltpu.async_copy(acc_hbm.at[sl_hi], b.at[pl.ds(0, size)], sem),
                pltpu.async_copy(cnt_hbm.at[sl_lo], c.at[pl.ds(0, size)], sem),
                pltpu.async_copy(cnt_hbm.at[sl_hi], d.at[pl.ds(0, size)], sem),
                pltpu.async_copy(srb_hbm.at[sl_lo], e.at[pl.ds(0, size)], sem),
            ]
            for h in handles:
                h.wait()

            @pl.loop(0, size, step=_L)
            def _(i):
                s = pl.ds(i, _L)
                cnt = jnp.maximum(c[s] + d[s], 1.0)
                a[s] = (a[s] + b[s]) / cnt + e[s]

            pltpu.sync_copy(a.at[pl.ds(0, size)], out_hbm.at[pl.ds(base, size)])

        @pl.when(wid < 31)
        def _():
            run(_FSTR)

        @pl.when(wid == 31)
        def _():
            run(_N - 31 * _FSTR)

    return k(acc2, cnt2, srb)


def kernel(x, seq, pause, edge_index, fc_W, fc_b, enc_W, enc_b, lin_l_W, lin_l_b, lin_r_W):
    ei = edge_index.astype(jnp.int32).reshape(2 * _E)
    sl, srb = _dense(x, pause, seq, enc_W, fc_W.reshape(32), fc_b, enc_b,
                     lin_l_W.reshape(64), lin_r_W.reshape(64), lin_l_b)
    acc2, cnt2 = _edges(sl, ei)
    out = _finalize(acc2, cnt2, srb)
    return out.reshape(_N, 1)


# speedup vs baseline: 1.2911x; 1.0323x over previous
import dataclasses
import functools

import jax
import jax.numpy as jnp
from jax import lax
from jax.experimental import pallas as pl
from jax.experimental.pallas import tpu as pltpu
from jax.experimental.pallas import tpu_sc as plsc

_N = 10000
_E = 320000
_SEQ_DIM = 9216
_NP = 10240
_BN = 512
_GRID = _NP // _BN
_NSUB = 16
_L = 16
_STR = _NP // _NSUB
_EPW = _E // 32
_UNROLL = 5
_BANKS = 3


def _gelu(t):
    return 0.5 * t * (1.0 + lax.erf(t * 0.7071067811865476))


def _dense_body(x_ref, pause_ref, seq_ref, encw_ref, fcw_ref, fcb_ref, encb_ref,
                wl_ref, wr_ref, linlb_ref, sl_ref, srb_ref):
    enc = lax.dot_general(seq_ref[...].astype(jnp.bfloat16),
                          encw_ref[...].astype(jnp.bfloat16),
                          dimension_numbers=(((1,), (1,)), ((), ())),
                          preferred_element_type=jnp.float32)
    fcw = fcw_ref[...]
    fcb = fcb_ref[...]
    wl = wl_ref[...]
    wr = wr_ref[...]
    g = _gelu(enc + encb_ref[...][None, :])
    h = x_ref[...][:, None] * fcw[None, :] + fcb[None, :] + g
    hp = pause_ref[...][:, None] * fcw[None, :] + fcb[None, :]
    fh = _gelu(h)
    fhp = _gelu(hp)
    sl_ref[...] = jnp.sum(fhp * wl[None, :32], axis=1) + jnp.sum(fh * wl[None, 32:], axis=1)
    srb_ref[...] = (jnp.sum(fhp * wr[None, :32], axis=1)
                    + jnp.sum(fh * wr[None, 32:], axis=1) + linlb_ref[0])


def _dense(x, pause, seq, encw, fcw, fcb, encb, wl, wr, linlb):
    full = lambda shape: pl.BlockSpec(shape, lambda i: tuple(0 for _ in shape))
    return pl.pallas_call(
        _dense_body,
        grid=(_GRID,),
        in_specs=[
            pl.BlockSpec((_BN,), lambda i: (i,)),
            pl.BlockSpec((_BN,), lambda i: (i,)),
            pl.BlockSpec((_BN, _SEQ_DIM), lambda i: (i, 0)),
            full((32, _SEQ_DIM)),
            full((32,)),
            full((32,)),
            full((32,)),
            full((64,)),
            full((64,)),
            full((1,)),
        ],
        out_specs=[pl.BlockSpec((_BN,), lambda i: (i,)),
                   pl.BlockSpec((_BN,), lambda i: (i,))],
        out_shape=[jax.ShapeDtypeStruct((_NP,), jnp.float32),
                   jax.ShapeDtypeStruct((_NP,), jnp.float32)],
    )(x, pause, seq, encw, fcw, fcb, encb, wl, wr, linlb)


def _sc_compiler_params():
    cp = pltpu.CompilerParams()
    if "needs_layout_passes" in pltpu.CompilerParams.__dataclass_fields__:
        cp = dataclasses.replace(cp, needs_layout_passes=False)
    return cp


_MESH = plsc.VectorSubcoreMesh(core_axis_name="c", subcore_axis_name="s")


def _stripe_reduce(part_all, stag, stripe, sid, sem):
    base = sid * _STR
    handles = [pltpu.async_copy(part_all.at[j, pl.ds(base, _STR)], stag.at[j], sem)
               for j in range(_NSUB)]
    for h in handles:
        h.wait()

    @pl.loop(0, _STR, step=_L)
    def _(k2):
        v = stag[0, pl.ds(k2, _L)]
        for j in range(1, _NSUB):
            v = v + stag[j, pl.ds(k2, _L)]
        stripe[pl.ds(k2, _L)] = v


def _edges(sl, eflat):
    @functools.partial(
        pl.kernel,
        mesh=_MESH,
        compiler_params=_sc_compiler_params(),
        out_type=[jax.ShapeDtypeStruct((2 * _NP,), jnp.float32),
                  jax.ShapeDtypeStruct((2 * _NP,), jnp.float32)],
        scratch_types=[
            pltpu.VMEM((_NP,), jnp.float32),
            pltpu.VMEM((_NP,), jnp.float32),
            pltpu.VMEM((_NP,), jnp.float32),
            pltpu.VMEM((_EPW,), jnp.int32),
            pltpu.VMEM((_EPW,), jnp.int32),
            pltpu.VMEM((_NSUB, _STR), jnp.float32),
            pltpu.VMEM((_STR,), jnp.float32),
            pltpu.VMEM_SHARED((_NSUB, _NP), jnp.float32),
            pltpu.VMEM_SHARED((_NSUB, _NP), jnp.float32),
            pltpu.VMEM_SHARED((_NP,), jnp.float32),
            pltpu.SemaphoreType.DMA,
        ],
    )
    def k(sl_hbm, e_hbm, acc_out, cnt_out,
          table, acc0, cnt0,
          srcb, dstb, stag, stripe, acc_all, cnt_all, sl_sh, sem):
        cid = lax.axis_index("c")
        sid = lax.axis_index("s")
        zv = jnp.zeros((_L,), jnp.float32)
        ones = jnp.ones((_L,), jnp.float32)
        ebase = (cid * _NSUB + sid) * _EPW
        h_tab = pltpu.async_copy(sl_hbm.at[pl.ds(sid * _STR, _STR)],
                                 sl_sh.at[pl.ds(sid * _STR, _STR)], sem)
        h_src = pltpu.async_copy(e_hbm.at[pl.ds(ebase, _EPW)], srcb, sem)
        h_dst = pltpu.async_copy(e_hbm.at[pl.ds(_E + ebase, _EPW)], dstb, sem)

        @pl.loop(0, _NP, step=_L)
        def _(i):
            acc0[pl.ds(i, _L)] = zv
            cnt0[pl.ds(i, _L)] = zv

        h_tab.wait()
        plsc.subcore_barrier()
        h_sh = pltpu.async_copy(sl_sh, table, sem)
        h_src.wait()
        h_dst.wait()
        h_sh.wait()

        @plsc.parallel_loop(0, _EPW, _L, unroll=_UNROLL)
        def _(i):
            sv = srcb[pl.ds(i, _L)]
            dv = dstb[pl.ds(i, _L)]
            vals = plsc.load_gather(table, [sv])
            plsc.addupdate_scatter(acc0, [dv], vals)
            plsc.addupdate_scatter(cnt0, [dv], ones)

        pltpu.sync_copy(acc0, acc_all.at[sid])
        pltpu.sync_copy(cnt0, cnt_all.at[sid])
        plsc.subcore_barrier()

        _stripe_reduce(acc_all, stag, stripe, sid, sem)
        pltpu.sync_copy(stripe, acc_out.at[pl.ds(cid * _NP + sid * _STR, _STR)])
        _stripe_reduce(cnt_all, stag, stripe, sid, sem)
        pltpu.sync_copy(stripe, cnt_out.at[pl.ds(cid * _NP + sid * _STR, _STR)])

    return k(sl, eflat)


_FSTR = 320


def _finalize(acc2, cnt2, srb):
    @functools.partial(
        pl.kernel,
        mesh=_MESH,
        compiler_params=_sc_compiler_params(),
        out_type=jax.ShapeDtypeStruct((_N,), jnp.float32),
        scratch_types=[
            pltpu.VMEM((_FSTR,), jnp.float32),
            pltpu.VMEM((_FSTR,), jnp.float32),
            pltpu.VMEM((_FSTR,), jnp.float32),
            pltpu.VMEM((_FSTR,), jnp.float32),
            pltpu.VMEM((_FSTR,), jnp.float32),
            pltpu.SemaphoreType.DMA,
        ],
    )
    def k(acc_hbm, cnt_hbm, srb_hbm, out_hbm, a, b, c, d, e, sem):
        cid = lax.axis_index("c")
        sid = lax.axis_index("s")
        wid = cid * _NSUB + sid
        base = wid * _FSTR

        def run(size):
            sl_lo = pl.ds(base, size)
            sl_hi = pl.ds(_NP + base, size)
            handles = [
                pltpu.async_copy(acc_hbm.at[sl_lo], a.at[pl.ds(0, size)], sem),
                pltpu.async_copy(acc_hbm.at[sl_hi], b.at[pl.ds(0, size)], sem),
                pltpu.async_copy(cnt_hbm.at[sl_lo], c.at[pl.ds(0, size)], sem),
                pltpu.async_copy(cnt_hbm.at[sl_hi], d.at[pl.ds(0, size)], sem),
                pltpu.async_copy(srb_hbm.at[sl_lo], e.at[pl.ds(0, size)], sem),
            ]
            for h in handles:
                h.wait()

            @pl.loop(0, size, step=_L)
            def _(i):
                s = pl.ds(i, _L)
                cnt = jnp.maximum(c[s] + d[s], 1.0)
                a[s] = (a[s] + b[s]) / cnt + e[s]

            pltpu.sync_copy(a.at[pl.ds(0, size)], out_hbm.at[pl.ds(base, size)])

        @pl.when(wid < 31)
        def _():
            run(_FSTR)

        @pl.when(wid == 31)
        def _():
            run(_N - 31 * _FSTR)

    return k(acc2, cnt2, srb)


def kernel(x, seq, pause, edge_index, fc_W, fc_b, enc_W, enc_b, lin_l_W, lin_l_b, lin_r_W):
    ei = edge_index.astype(jnp.int32).reshape(2 * _E)
    sl, srb = _dense(x, pause, seq, enc_W, fc_W.reshape(32), fc_b, enc_b,
                     lin_l_W.reshape(64), lin_r_W.reshape(64), lin_l_b)
    acc2, cnt2 = _edges(sl, ei)
    out = _finalize(acc2, cnt2, srb)
    return out.reshape(_N, 1)


# speedup vs baseline: 1.2954x; 1.0033x over previous
import dataclasses
import functools

import jax
import jax.numpy as jnp
from jax import lax
from jax.experimental import pallas as pl
from jax.experimental.pallas import tpu as pltpu
from jax.experimental.pallas import tpu_sc as plsc

_N = 10000
_E = 320000
_SEQ_DIM = 9216
_NP = 10240
_BN = 512
_GRID = _NP // _BN
_NSUB = 16
_L = 16
_STR = _NP // _NSUB
_EPW = _E // 32
_UNROLL = 5


def _gelu(t):
    return 0.5 * t * (1.0 + lax.erf(t * 0.7071067811865476))


def _dense_body(x_ref, pause_ref, seq_ref, encw_ref, fcw_ref, fcb_ref, encb_ref,
                wl_ref, wr_ref, linlb_ref, sl_ref, srb_ref):
    enc = lax.dot_general(seq_ref[...].astype(jnp.bfloat16),
                          encw_ref[...].astype(jnp.bfloat16),
                          dimension_numbers=(((1,), (1,)), ((), ())),
                          preferred_element_type=jnp.float32)
    fcw = fcw_ref[...]
    fcb = fcb_ref[...]
    wl = wl_ref[...]
    wr = wr_ref[...]
    g = _gelu(enc + encb_ref[...][None, :])
    h = x_ref[...][:, None] * fcw[None, :] + fcb[None, :] + g
    hp = pause_ref[...][:, None] * fcw[None, :] + fcb[None, :]
    fh = _gelu(h)
    fhp = _gelu(hp)
    sl_ref[...] = jnp.sum(fhp * wl[None, :32], axis=1) + jnp.sum(fh * wl[None, 32:], axis=1)
    srb_ref[...] = (jnp.sum(fhp * wr[None, :32], axis=1)
                    + jnp.sum(fh * wr[None, 32:], axis=1) + linlb_ref[0])


def _dense(x, pause, seq, encw, fcw, fcb, encb, wl, wr, linlb):
    full = lambda shape: pl.BlockSpec(shape, lambda i: tuple(0 for _ in shape))
    return pl.pallas_call(
        _dense_body,
        grid=(_GRID,),
        in_specs=[
            pl.BlockSpec((_BN,), lambda i: (i,)),
            pl.BlockSpec((_BN,), lambda i: (i,)),
            pl.BlockSpec((_BN, _SEQ_DIM), lambda i: (i, 0)),
            full((32, _SEQ_DIM)),
            full((32,)),
            full((32,)),
            full((32,)),
            full((64,)),
            full((64,)),
            full((1,)),
        ],
        out_specs=[pl.BlockSpec((_BN,), lambda i: (i,)),
                   pl.BlockSpec((_BN,), lambda i: (i,))],
        out_shape=[jax.ShapeDtypeStruct((_NP,), jnp.float32),
                   jax.ShapeDtypeStruct((_NP,), jnp.float32)],
    )(x, pause, seq, encw, fcw, fcb, encb, wl, wr, linlb)


def _sc_compiler_params():
    cp = pltpu.CompilerParams()
    if "needs_layout_passes" in pltpu.CompilerParams.__dataclass_fields__:
        cp = dataclasses.replace(cp, needs_layout_passes=False)
    return cp


_MESH = plsc.VectorSubcoreMesh(core_axis_name="c", subcore_axis_name="s")


def _stripe_reduce(part_all, stag, stripe, sid, sem):
    base = sid * _STR
    handles = [pltpu.async_copy(part_all.at[j, pl.ds(base, _STR)], stag.at[j], sem)
               for j in range(_NSUB)]
    for h in handles:
        h.wait()

    @pl.loop(0, _STR, step=_L)
    def _(k2):
        v = stag[0, pl.ds(k2, _L)]
        for j in range(1, _NSUB):
            v = v + stag[j, pl.ds(k2, _L)]
        stripe[pl.ds(k2, _L)] = v


def _edges(sl, eflat):
    @functools.partial(
        pl.kernel,
        mesh=_MESH,
        compiler_params=_sc_compiler_params(),
        out_type=[jax.ShapeDtypeStruct((2 * _NP,), jnp.float32),
                  jax.ShapeDtypeStruct((2 * _NP,), jnp.float32)],
        scratch_types=[
            pltpu.VMEM((_NP,), jnp.float32),
            pltpu.VMEM((_NP,), jnp.float32),
            pltpu.VMEM((_NP,), jnp.float32),
            pltpu.VMEM((_EPW,), jnp.int32),
            pltpu.VMEM((_EPW,), jnp.int32),
            pltpu.VMEM((_NSUB, _STR), jnp.float32),
            pltpu.VMEM((_STR,), jnp.float32),
            pltpu.VMEM_SHARED((_NSUB, _NP), jnp.float32),
            pltpu.VMEM_SHARED((_NSUB, _NP), jnp.float32),
            pltpu.VMEM_SHARED((_NP,), jnp.float32),
            pltpu.SemaphoreType.DMA,
        ],
    )
    def k(sl_hbm, e_hbm, acc_out, cnt_out,
          table, acc0, cnt0,
          srcb, dstb, stag, stripe, acc_all, cnt_all, sl_sh, sem):
        cid = lax.axis_index("c")
        sid = lax.axis_index("s")
        zv = jnp.zeros((_L,), jnp.float32)
        ones = jnp.ones((_L,), jnp.float32)
        ebase = (cid * _NSUB + sid) * _EPW
        h_tab = pltpu.async_copy(sl_hbm.at[pl.ds(sid * _STR, _STR)],
                                 sl_sh.at[pl.ds(sid * _STR, _STR)], sem)
        h_src = pltpu.async_copy(e_hbm.at[pl.ds(ebase, _EPW)], srcb, sem)
        h_dst = pltpu.async_copy(e_hbm.at[pl.ds(_E + ebase, _EPW)], dstb, sem)

        @pl.loop(0, _NP, step=_L)
        def _(i):
            acc0[pl.ds(i, _L)] = zv
            cnt0[pl.ds(i, _L)] = zv

        h_tab.wait()
        plsc.subcore_barrier()
        h_sh = pltpu.async_copy(sl_sh, table, sem)
        h_src.wait()
        h_dst.wait()
        h_sh.wait()

        @plsc.parallel_loop(0, _EPW, _L, unroll=_UNROLL)
        def _(i):
            sv = srcb[pl.ds(i, _L)]
            dv = dstb[pl.ds(i, _L)]
            vals = plsc.load_gather(table, [sv])
            plsc.addupdate_scatter(acc0, [dv], vals)
            plsc.addupdate_scatter(cnt0, [dv], ones)

        pltpu.sync_copy(acc0, acc_all.at[sid])
        pltpu.sync_copy(cnt0, cnt_all.at[sid])
        plsc.subcore_barrier()

        _stripe_reduce(acc_all, stag, stripe, sid, sem)
        pltpu.sync_copy(stripe, acc_out.at[pl.ds(cid * _NP + sid * _STR, _STR)])
        _stripe_reduce(cnt_all, stag, stripe, sid, sem)
        pltpu.sync_copy(stripe, cnt_out.at[pl.ds(cid * _NP + sid * _STR, _STR)])

    return k(sl, eflat)


_FSTR = 320


def _finalize(acc2, cnt2, srb):
    @functools.partial(
        pl.kernel,
        mesh=_MESH,
        compiler_params=_sc_compiler_params(),
        out_type=jax.ShapeDtypeStruct((_N,), jnp.float32),
        scratch_types=[
            pltpu.VMEM((_FSTR,), jnp.float32),
            pltpu.VMEM((_FSTR,), jnp.float32),
            pltpu.VMEM((_FSTR,), jnp.float32),
            pltpu.VMEM((_FSTR,), jnp.float32),
            pltpu.VMEM((_FSTR,), jnp.float32),
            pltpu.SemaphoreType.DMA,
        ],
    )
    def k(acc_hbm, cnt_hbm, srb_hbm, out_hbm, a, b, c, d, e, sem):
        cid = lax.axis_index("c")
        sid = lax.axis_index("s")
        wid = cid * _NSUB + sid
        base = wid * _FSTR

        def run(size):
            sl_lo = pl.ds(base, size)
            sl_hi = pl.ds(_NP + base, size)
            handles = [
                pltpu.async_copy(acc_hbm.at[sl_lo], a.at[pl.ds(0, size)], sem),
                pltpu.async_copy(acc_hbm.at[sl_hi], b.at[pl.ds(0, size)], sem),
                pltpu.async_copy(cnt_hbm.at[sl_lo], c.at[pl.ds(0, size)], sem),
                pltpu.async_copy(cnt_hbm.at[sl_hi], d.at[pl.ds(0, size)], sem),
                pltpu.async_copy(srb_hbm.at[sl_lo], e.at[pl.ds(0, size)], sem),
            ]
            for h in handles:
                h.wait()

            @pl.loop(0, size, step=_L)
            def _(i):
                s = pl.ds(i, _L)
                cnt = jnp.maximum(c[s] + d[s], 1.0)
                a[s] = (a[s] + b[s]) / cnt + e[s]

            pltpu.sync_copy(a.at[pl.ds(0, size)], out_hbm.at[pl.ds(base, size)])

        @pl.when(wid < 31)
        def _():
            run(_FSTR)

        @pl.when(wid == 31)
        def _():
            run(_N - 31 * _FSTR)

    return k(acc2, cnt2, srb)


def kernel(x, seq, pause, edge_index, fc_W, fc_b, enc_W, enc_b, lin_l_W, lin_l_b, lin_r_W):
    ei = edge_index.astype(jnp.int32).reshape(2 * _E)
    sl, srb = _dense(x, pause, seq, enc_W, fc_W.reshape(32), fc_b, enc_b,
                     lin_l_W.reshape(64), lin_r_W.reshape(64), lin_l_b)
    acc2, cnt2 = _edges(sl, ei)
    out = _finalize(acc2, cnt2, srb)
    return out.reshape(_N, 1)
